# baseline (speedup 1.0000x reference)
import functools

import jax
import jax.numpy as jnp
from jax import lax
from jax.experimental import pallas as pl
from jax.experimental.pallas import tpu as pltpu
from jax.experimental.pallas import tpu_sc as plsc

_NUM_CORES = 2
_NUM_SUBCORES = 16
_NW = _NUM_CORES * _NUM_SUBCORES
_LANES = 16
_CHUNK = 128
_NSUMS = 6


def _sc_scores_body(ids_hbm, attn_hbm, tt_hbm, tfidf_hbm, spec_hbm, dflt_hbm,
                    out_hbm, part_hbm,
                    idx_v, attn_v, tt_v, vals_v, spec_v, dflt_v, pbuf_v,
                    gsem, hsem):
    n_tok = ids_hbm.shape[0]
    per_w = n_tok // _NW
    n_chunks = per_w // _CHUNK
    wid = lax.axis_index("s") * _NUM_CORES + lax.axis_index("c")
    base = wid * per_w

    others = [
        pltpu.async_copy(attn_hbm.at[pl.ds(base, per_w)], attn_v, hsem),
        pltpu.async_copy(tt_hbm.at[pl.ds(base, per_w)], tt_v, hsem),
        pltpu.async_copy(spec_hbm, spec_v, hsem),
        pltpu.async_copy(dflt_hbm, dflt_v, hsem),
    ]
    pltpu.sync_copy(ids_hbm.at[pl.ds(base, per_w)], idx_v)
    gathers = [
        pltpu.async_copy(
            tfidf_hbm.at[idx_v.at[pl.ds(j * _CHUNK, _CHUNK)]],
            vals_v.at[pl.ds(j * _CHUNK, _CHUNK)],
            gsem,
        )
        for j in range(n_chunks)
    ]
    for c in gathers:
        c.wait()
    for c in others:
        c.wait()

    s0 = spec_v[pl.ds(0, _LANES)]
    s1 = spec_v[pl.ds(_LANES, _LANES)]
    s2 = spec_v[pl.ds(2 * _LANES, _LANES)]
    s3 = spec_v[pl.ds(3 * _LANES, _LANES)]
    dflt = dflt_v[...]
    one = jnp.ones((_LANES,), jnp.float32)
    zero = jnp.zeros((_LANES,), jnp.float32)
    acc = [zero] * _NSUMS
    for t in range(per_w // _LANES):
        sl = pl.ds(t * _LANES, _LANES)
        tok = idx_v[sl]
        val = vals_v[sl]
        amask = attn_v[sl]
        tt = tt_v[sl]
        is_special = (tok == s0) | (tok == s1) | (tok == s2) | (tok == s3)
        score = jnp.where(is_special, dflt, val) * amask
        vals_v[sl] = score
        ctx = jnp.where(tt == 1, one, zero)
        cmt = jnp.where(tt == 0, one, zero)
        acc[0] = acc[0] + score
        acc[1] = acc[1] + amask
        acc[2] = acc[2] + score * ctx
        acc[3] = acc[3] + ctx * amask
        acc[4] = acc[4] + score * cmt
        acc[5] = acc[5] + cmt * amask

    for k in range(_NSUMS):
        pbuf_v[pl.ds(k * _LANES, _LANES)] = acc[k]
    pbuf_v[pl.ds(_NSUMS * _LANES, _LANES)] = zero
    pbuf_v[pl.ds((_NSUMS + 1) * _LANES, _LANES)] = zero

    w0 = pltpu.async_copy(vals_v, out_hbm.at[pl.ds(base, per_w)], gsem)
    w1 = pltpu.async_copy(pbuf_v, part_hbm.at[pl.ds(wid * 128, 128)], hsem)
    w0.wait()
    w1.wait()


def _sc_scores(ids_flat, attn_flat, tt_flat, tfidf, spec_vec, dflt_vec):
    n_tok = ids_flat.shape[0]
    per_w = n_tok // _NW
    call = functools.partial(
        pl.kernel,
        mesh=plsc.VectorSubcoreMesh(core_axis_name="c", subcore_axis_name="s"),
        out_type=[
            jax.ShapeDtypeStruct((n_tok,), jnp.float32),
            jax.ShapeDtypeStruct((_NW * 128,), jnp.float32),
        ],
        scratch_types=[
            pltpu.VMEM((per_w,), jnp.int32),
            pltpu.VMEM((per_w,), jnp.float32),
            pltpu.VMEM((per_w,), jnp.int32),
            pltpu.VMEM((per_w,), jnp.float32),
            pltpu.VMEM((4 * _LANES,), jnp.int32),
            pltpu.VMEM((_LANES,), jnp.float32),
            pltpu.VMEM((128,), jnp.float32),
            pltpu.SemaphoreType.DMA,
            pltpu.SemaphoreType.DMA,
        ],
    )(_sc_scores_body)
    return call(ids_flat, attn_flat, tt_flat, tfidf, spec_vec, dflt_vec)


def _tc_body(n_blocks, emb_ref, sc_ref, part_ref, out_ref,
             mean_ref, ctx_ref, cmt_ref):
    i = pl.program_id(0)
    blk = emb_ref.shape[0]
    col = sc_ref[...].reshape(blk, 1)
    out_ref[...] = emb_ref[...] * col

    @pl.when(i == n_blocks - 1)
    def _final():
        mean_ref[...] = jnp.zeros((1, 1), jnp.float32)
        ctx_ref[...] = jnp.zeros((1, 1), jnp.float32)
        cmt_ref[...] = jnp.zeros((1, 1), jnp.float32)


def _tc_multiply(emb2d, scores_rows, partials, blk):
    n, d = emb2d.shape
    rows = partials.shape[0]
    n_blocks = n // blk
    scalar_spec = pl.BlockSpec((1, 1), lambda i: (0, 0))
    return pl.pallas_call(
        functools.partial(_tc_body, n_blocks),
        grid=(n_blocks,),
        in_specs=[
            pl.BlockSpec((blk, d), lambda i: (i, 0)),
            pl.BlockSpec((1, 1, blk), lambda i: (i, 0, 0)),
            pl.BlockSpec((rows, 128), lambda i: (0, 0)),
        ],
        out_specs=[
            pl.BlockSpec((blk, d), lambda i: (i, 0)),
            scalar_spec,
            scalar_spec,
            scalar_spec,
        ],
        out_shape=[
            jax.ShapeDtypeStruct((n, d), jnp.float32),
            jax.ShapeDtypeStruct((1, 1), jnp.float32),
            jax.ShapeDtypeStruct((1, 1), jnp.float32),
            jax.ShapeDtypeStruct((1, 1), jnp.float32),
        ],
        compiler_params=pltpu.CompilerParams(
            dimension_semantics=("parallel",),
        ),
    )(emb2d, scores_rows, partials)


def kernel(embeddings, input_ids, token_type_ids, attention_mask,
           special_token_ids, tfidf_scores, default_score):
    b, l, d = embeddings.shape
    n = b * l

    ids_flat = input_ids.reshape(n).astype(jnp.int32)
    attn_flat = attention_mask.reshape(n).astype(jnp.float32)
    tt_flat = token_type_ids.reshape(n).astype(jnp.int32)
    sp = special_token_ids.astype(jnp.int32)
    spec_vec = jnp.repeat(sp, _LANES)
    dflt_vec = jnp.full((_LANES,), default_score, jnp.float32)

    tfidf = tfidf_scores.astype(jnp.float32)
    emb2d = embeddings.reshape(n, d)

    scores, parts = _sc_scores(ids_flat, attn_flat, tt_flat,
                               tfidf, spec_vec, dflt_vec)

    blk = 2048
    masked, mean_v, ctx_v, cmt_v = _tc_multiply(
        emb2d, scores.reshape(n // blk, 1, blk), parts.reshape(_NW, 128),
        blk=blk)

    p = parts.reshape(_NW, 8, _LANES).sum(axis=(0, 2))
    return (
        masked.reshape(b, l, d),
        scores.reshape(b, l, 1),
        mean_v[0, 0] + p[0] / (p[1] + 1e-8),
        ctx_v[0, 0] + p[2] / (p[3] + 1e-8),
        cmt_v[0, 0] + p[4] / (p[5] + 1e-8),
    )


# speedup vs baseline: 1.0928x; 1.0928x over previous
import functools

import jax
import jax.numpy as jnp
from jax import lax
from jax.experimental import pallas as pl
from jax.experimental.pallas import tpu as pltpu
from jax.experimental.pallas import tpu_sc as plsc

_NUM_CORES = 2
_NUM_SUBCORES = 16
_NW = _NUM_CORES * _NUM_SUBCORES
_LANES = 16
_CHUNK = 128
_NSUMS = 6


def _sc_scores_body(ids_hbm, attn_hbm, tt_hbm, tfidf_hbm, spec_hbm, dflt_hbm,
                    out_hbm, part_hbm,
                    idx_v, attn_v, tt_v, vals_v, spec_v, dflt_v, pbuf_v,
                    g0, g1, g2, g3, g4, g5, g6, g7, hsem, wsem):
    n_tok = ids_hbm.shape[0]
    per_w = n_tok // _NW
    n_chunks = per_w // _CHUNK
    gsems = [g0, g1, g2, g3, g4, g5, g6, g7]
    assert n_chunks == len(gsems)
    wid = lax.axis_index("s") * _NUM_CORES + lax.axis_index("c")
    base = wid * per_w

    others = [
        pltpu.async_copy(attn_hbm.at[pl.ds(base, per_w)], attn_v, hsem),
        pltpu.async_copy(tt_hbm.at[pl.ds(base, per_w)], tt_v, hsem),
        pltpu.async_copy(spec_hbm, spec_v, hsem),
        pltpu.async_copy(dflt_hbm, dflt_v, hsem),
    ]
    pltpu.sync_copy(ids_hbm.at[pl.ds(base, per_w)], idx_v)
    gathers = [
        pltpu.async_copy(
            tfidf_hbm.at[idx_v.at[pl.ds(j * _CHUNK, _CHUNK)]],
            vals_v.at[pl.ds(j * _CHUNK, _CHUNK)],
            gsems[j],
        )
        for j in range(n_chunks)
    ]
    for c in others:
        c.wait()

    s0 = spec_v[pl.ds(0, _LANES)]
    s1 = spec_v[pl.ds(_LANES, _LANES)]
    s2 = spec_v[pl.ds(2 * _LANES, _LANES)]
    s3 = spec_v[pl.ds(3 * _LANES, _LANES)]
    dflt = dflt_v[...]
    one = jnp.ones((_LANES,), jnp.float32)
    zero = jnp.zeros((_LANES,), jnp.float32)
    acc = [zero] * _NSUMS
    writes = []
    for j in range(n_chunks):
        gathers[j].wait()
        for t in range(j * _CHUNK // _LANES, (j + 1) * _CHUNK // _LANES):
            sl = pl.ds(t * _LANES, _LANES)
            tok = idx_v[sl]
            val = vals_v[sl]
            amask = attn_v[sl]
            tt = tt_v[sl]
            is_special = (tok == s0) | (tok == s1) | (tok == s2) | (tok == s3)
            score = jnp.where(is_special, dflt, val) * amask
            vals_v[sl] = score
            ctx = jnp.where(tt == 1, one, zero)
            cmt = jnp.where(tt == 0, one, zero)
            acc[0] = acc[0] + score
            acc[1] = acc[1] + amask
            acc[2] = acc[2] + score * ctx
            acc[3] = acc[3] + ctx * amask
            acc[4] = acc[4] + score * cmt
            acc[5] = acc[5] + cmt * amask
        writes.append(pltpu.async_copy(
            vals_v.at[pl.ds(j * _CHUNK, _CHUNK)],
            out_hbm.at[pl.ds(base + j * _CHUNK, _CHUNK)],
            wsem,
        ))

    for k in range(_NSUMS):
        pbuf_v[pl.ds(k * _LANES, _LANES)] = acc[k]
    pbuf_v[pl.ds(_NSUMS * _LANES, _LANES)] = zero
    pbuf_v[pl.ds((_NSUMS + 1) * _LANES, _LANES)] = zero

    w1 = pltpu.async_copy(pbuf_v, part_hbm.at[pl.ds(wid * 128, 128)], hsem)
    for c in writes:
        c.wait()
    w1.wait()


def _sc_scores(ids_flat, attn_flat, tt_flat, tfidf, spec_vec, dflt_vec):
    n_tok = ids_flat.shape[0]
    per_w = n_tok // _NW
    call = functools.partial(
        pl.kernel,
        mesh=plsc.VectorSubcoreMesh(core_axis_name="c", subcore_axis_name="s"),
        out_type=[
            jax.ShapeDtypeStruct((n_tok,), jnp.float32),
            jax.ShapeDtypeStruct((_NW * 128,), jnp.float32),
        ],
        scratch_types=[
            pltpu.VMEM((per_w,), jnp.int32),
            pltpu.VMEM((per_w,), jnp.float32),
            pltpu.VMEM((per_w,), jnp.int32),
            pltpu.VMEM((per_w,), jnp.float32),
            pltpu.VMEM((4 * _LANES,), jnp.int32),
            pltpu.VMEM((_LANES,), jnp.float32),
            pltpu.VMEM((128,), jnp.float32),
        ] + [pltpu.SemaphoreType.DMA] * 10,
    )(_sc_scores_body)
    return call(ids_flat, attn_flat, tt_flat, tfidf, spec_vec, dflt_vec)


def _tc_body(n_blocks, emb_ref, sc_ref, part_ref, out_ref,
             mean_ref, ctx_ref, cmt_ref):
    i = pl.program_id(0)
    blk = emb_ref.shape[0]
    col = sc_ref[...].reshape(blk, 1)
    out_ref[...] = emb_ref[...] * col

    @pl.when(i == n_blocks - 1)
    def _final():
        p = part_ref[...]
        lane = lax.broadcasted_iota(jnp.int32, p.shape, 1) // _LANES
        sums = [jnp.sum(jnp.where(lane == k, p, 0.0)) for k in range(_NSUMS)]
        mean_ref[...] = jnp.full((1, 1), sums[0] / (sums[1] + 1e-8),
                                 jnp.float32)
        ctx_ref[...] = jnp.full((1, 1), sums[2] / (sums[3] + 1e-8),
                                jnp.float32)
        cmt_ref[...] = jnp.full((1, 1), sums[4] / (sums[5] + 1e-8),
                                jnp.float32)


def _tc_multiply(emb2d, scores_rows, partials, blk):
    n, d = emb2d.shape
    rows = partials.shape[0]
    n_blocks = n // blk
    scalar_spec = pl.BlockSpec((1, 1), lambda i: (0, 0))
    return pl.pallas_call(
        functools.partial(_tc_body, n_blocks),
        grid=(n_blocks,),
        in_specs=[
            pl.BlockSpec((blk, d), lambda i: (i, 0)),
            pl.BlockSpec((1, 1, blk), lambda i: (i, 0, 0)),
            pl.BlockSpec((rows, 128), lambda i: (0, 0)),
        ],
        out_specs=[
            pl.BlockSpec((blk, d), lambda i: (i, 0)),
            scalar_spec,
            scalar_spec,
            scalar_spec,
        ],
        out_shape=[
            jax.ShapeDtypeStruct((n, d), jnp.float32),
            jax.ShapeDtypeStruct((1, 1), jnp.float32),
            jax.ShapeDtypeStruct((1, 1), jnp.float32),
            jax.ShapeDtypeStruct((1, 1), jnp.float32),
        ],
        compiler_params=pltpu.CompilerParams(
            dimension_semantics=("parallel",),
        ),
    )(emb2d, scores_rows, partials)


def kernel(embeddings, input_ids, token_type_ids, attention_mask,
           special_token_ids, tfidf_scores, default_score):
    b, l, d = embeddings.shape
    n = b * l

    ids_flat = input_ids.reshape(n).astype(jnp.int32)
    attn_flat = attention_mask.reshape(n).astype(jnp.float32)
    tt_flat = token_type_ids.reshape(n).astype(jnp.int32)
    sp = special_token_ids.astype(jnp.int32)
    spec_vec = jnp.repeat(sp, _LANES)
    dflt_vec = jnp.full((_LANES,), default_score, jnp.float32)

    tfidf = tfidf_scores.astype(jnp.float32)
    emb2d = embeddings.reshape(n, d)

    scores, parts = _sc_scores(ids_flat, attn_flat, tt_flat,
                               tfidf, spec_vec, dflt_vec)

    blk = 2048
    masked, mean_v, ctx_v, cmt_v = _tc_multiply(
        emb2d, scores.reshape(n // blk, 1, blk), parts.reshape(_NW, 128),
        blk=blk)

    return (
        masked.reshape(b, l, d),
        scores.reshape(b, l, 1),
        mean_v[0, 0],
        ctx_v[0, 0],
        cmt_v[0, 0],
    )


# speedup vs baseline: 1.0968x; 1.0037x over previous
import functools

import jax
import jax.numpy as jnp
from jax import lax
from jax.experimental import pallas as pl
from jax.experimental.pallas import tpu as pltpu
from jax.experimental.pallas import tpu_sc as plsc

_NUM_CORES = 2
_NUM_SUBCORES = 16
_NW = _NUM_CORES * _NUM_SUBCORES
_LANES = 16
_CHUNK = 128
_NSUMS = 6


def _sc_scores_body(ids_hbm, attn_hbm, tt_hbm, tfidf_hbm, spec_hbm, dflt_hbm,
                    out_hbm, part_hbm,
                    idx_v, attn_v, tt_v, vals_v, spec_v, dflt_v, pbuf_v,
                    gsem, hsem):
    n_tok = ids_hbm.shape[0]
    per_w = n_tok // _NW
    n_chunks = per_w // _CHUNK
    wid = lax.axis_index("s") * _NUM_CORES + lax.axis_index("c")
    base = wid * per_w

    others = [
        pltpu.async_copy(attn_hbm.at[pl.ds(base, per_w)], attn_v, hsem),
        pltpu.async_copy(tt_hbm.at[pl.ds(base, per_w)], tt_v, hsem),
        pltpu.async_copy(spec_hbm, spec_v, hsem),
        pltpu.async_copy(dflt_hbm, dflt_v, hsem),
    ]
    pltpu.sync_copy(ids_hbm.at[pl.ds(base, per_w)], idx_v)
    gathers = [
        pltpu.async_copy(
            tfidf_hbm.at[idx_v.at[pl.ds(j * _CHUNK, _CHUNK)]],
            vals_v.at[pl.ds(j * _CHUNK, _CHUNK)],
            gsem,
        )
        for j in range(n_chunks)
    ]
    for c in gathers:
        c.wait()
    for c in others:
        c.wait()

    s0 = spec_v[pl.ds(0, _LANES)]
    s1 = spec_v[pl.ds(_LANES, _LANES)]
    s2 = spec_v[pl.ds(2 * _LANES, _LANES)]
    s3 = spec_v[pl.ds(3 * _LANES, _LANES)]
    dflt = dflt_v[...]
    one = jnp.ones((_LANES,), jnp.float32)
    zero = jnp.zeros((_LANES,), jnp.float32)
    acc = [zero] * _NSUMS
    for t in range(per_w // _LANES):
        sl = pl.ds(t * _LANES, _LANES)
        tok = idx_v[sl]
        val = vals_v[sl]
        amask = attn_v[sl]
        tt = tt_v[sl]
        is_special = (tok == s0) | (tok == s1) | (tok == s2) | (tok == s3)
        score = jnp.where(is_special, dflt, val) * amask
        vals_v[sl] = score
        ctx = jnp.where(tt == 1, one, zero)
        cmt = jnp.where(tt == 0, one, zero)
        acc[0] = acc[0] + score
        acc[1] = acc[1] + amask
        acc[2] = acc[2] + score * ctx
        acc[3] = acc[3] + ctx * amask
        acc[4] = acc[4] + score * cmt
        acc[5] = acc[5] + cmt * amask

    for k in range(_NSUMS):
        pbuf_v[pl.ds(k * _LANES, _LANES)] = acc[k]
    pbuf_v[pl.ds(_NSUMS * _LANES, _LANES)] = zero
    pbuf_v[pl.ds((_NSUMS + 1) * _LANES, _LANES)] = zero

    w0 = pltpu.async_copy(vals_v, out_hbm.at[pl.ds(base, per_w)], gsem)
    w1 = pltpu.async_copy(pbuf_v, part_hbm.at[pl.ds(wid * 128, 128)], hsem)
    w0.wait()
    w1.wait()


def _sc_scores(ids_flat, attn_flat, tt_flat, tfidf, spec_vec, dflt_vec):
    n_tok = ids_flat.shape[0]
    per_w = n_tok // _NW
    call = functools.partial(
        pl.kernel,
        mesh=plsc.VectorSubcoreMesh(core_axis_name="c", subcore_axis_name="s"),
        out_type=[
            jax.ShapeDtypeStruct((n_tok,), jnp.float32),
            jax.ShapeDtypeStruct((_NW * 128,), jnp.float32),
        ],
        scratch_types=[
            pltpu.VMEM((per_w,), jnp.int32),
            pltpu.VMEM((per_w,), jnp.float32),
            pltpu.VMEM((per_w,), jnp.int32),
            pltpu.VMEM((per_w,), jnp.float32),
            pltpu.VMEM((4 * _LANES,), jnp.int32),
            pltpu.VMEM((_LANES,), jnp.float32),
            pltpu.VMEM((128,), jnp.float32),
            pltpu.SemaphoreType.DMA,
            pltpu.SemaphoreType.DMA,
        ],
    )(_sc_scores_body)
    return call(ids_flat, attn_flat, tt_flat, tfidf, spec_vec, dflt_vec)


def _tc_body(n_blocks, emb_ref, sc_ref, part_ref, out_ref,
             mean_ref, ctx_ref, cmt_ref):
    i = pl.program_id(0)
    blk = emb_ref.shape[0]
    col = sc_ref[i].reshape(blk, 1)
    out_ref[...] = emb_ref[...] * col

    @pl.when(i == n_blocks - 1)
    def _final():
        p = part_ref[...]
        lane = lax.broadcasted_iota(jnp.int32, p.shape, 1) // _LANES
        sums = [jnp.sum(jnp.where(lane == k, p, 0.0)) for k in range(_NSUMS)]
        mean_ref[...] = jnp.full((1, 1), sums[0] / (sums[1] + 1e-8),
                                 jnp.float32)
        ctx_ref[...] = jnp.full((1, 1), sums[2] / (sums[3] + 1e-8),
                                jnp.float32)
        cmt_ref[...] = jnp.full((1, 1), sums[4] / (sums[5] + 1e-8),
                                jnp.float32)


def _tc_multiply(emb2d, scores_rows, partials, blk):
    n, d = emb2d.shape
    rows = partials.shape[0]
    n_blocks = n // blk
    scalar_spec = pl.BlockSpec((1, 1), lambda i: (0, 0))
    return pl.pallas_call(
        functools.partial(_tc_body, n_blocks),
        grid=(n_blocks,),
        in_specs=[
            pl.BlockSpec((blk, d), lambda i: (i, 0)),
            pl.BlockSpec((n_blocks, 1, blk), lambda i: (0, 0, 0)),
            pl.BlockSpec((rows, 128), lambda i: (0, 0)),
        ],
        out_specs=[
            pl.BlockSpec((blk, d), lambda i: (i, 0)),
            scalar_spec,
            scalar_spec,
            scalar_spec,
        ],
        out_shape=[
            jax.ShapeDtypeStruct((n, d), jnp.float32),
            jax.ShapeDtypeStruct((1, 1), jnp.float32),
            jax.ShapeDtypeStruct((1, 1), jnp.float32),
            jax.ShapeDtypeStruct((1, 1), jnp.float32),
        ],
        compiler_params=pltpu.CompilerParams(
            dimension_semantics=("parallel",),
        ),
    )(emb2d, scores_rows, partials)


def kernel(embeddings, input_ids, token_type_ids, attention_mask,
           special_token_ids, tfidf_scores, default_score):
    b, l, d = embeddings.shape
    n = b * l

    ids_flat = input_ids.reshape(n).astype(jnp.int32)
    attn_flat = attention_mask.reshape(n).astype(jnp.float32)
    tt_flat = token_type_ids.reshape(n).astype(jnp.int32)
    sp = special_token_ids.astype(jnp.int32)
    spec_vec = jnp.repeat(sp, _LANES)
    dflt_vec = jnp.full((_LANES,), default_score, jnp.float32)

    tfidf = tfidf_scores.astype(jnp.float32)
    emb2d = embeddings.reshape(n, d)

    scores, parts = _sc_scores(ids_flat, attn_flat, tt_flat,
                               tfidf, spec_vec, dflt_vec)

    blk = 2048
    masked, mean_v, ctx_v, cmt_v = _tc_multiply(
        emb2d, scores.reshape(n // blk, 1, blk), parts.reshape(_NW, 128),
        blk=blk)

    return (
        masked.reshape(b, l, d),
        scores.reshape(b, l, 1),
        mean_v[0, 0],
        ctx_v[0, 0],
        cmt_v[0, 0],
    )
